# Initial kernel scaffold; baseline (speedup 1.0000x reference)
import jax
import jax.numpy as jnp
from jax.experimental import pallas as pl


def kernel(x, edge_index, W1, b1, W2, b2):
    raise NotImplementedError("write your pallas kernel here")


# speedup vs baseline: 15.2683x; 15.2683x over previous
import functools

import jax
import jax.numpy as jnp
from jax import lax
from jax.experimental import pallas as pl
from jax.experimental.pallas import tpu as pltpu
from jax.experimental.pallas import tpu_sc as plsc

N_NODES = 10000
N_EDGES = 320000
IN_DIM = 128
HID_DIM = 32
EMB_DIM = 64

NC = 2
NS = 16
NW = NC * NS
E_W = N_EDGES // NW
B = 80
NB = E_W // B
NP = 10240
RPS = NP // NS
ZB = 128

_mesh = plsc.VectorSubcoreMesh(
    core_axis_name="c", subcore_axis_name="s", num_cores=NC, num_subcores=NS
)

_sc_params = pltpu.CompilerParams(use_tc_tiling_on_sc=False)

f32 = jnp.float32
i32 = jnp.int32


def _zero_fill(zbuf, f):
    z = jnp.zeros((16,), f32)

    @pl.loop(0, ZB)
    def _(r):
        for j in range(f // 16):
            zbuf[r, pl.ds(j * 16, 16)] = z


def _zero_acc(zbuf, acc, sid, f):
    _zero_fill(zbuf, f)
    for k in range(RPS // ZB):
        pltpu.sync_copy(zbuf, acc.at[pl.ds(sid * RPS + k * ZB, ZB)])


def _copy_out(acc, out_hbm, cid, sid):
    pltpu.sync_copy(
        acc.at[pl.ds(sid * RPS, RPS)], out_hbm.at[cid, pl.ds(sid * RPS, RPS)]
    )


@functools.partial(
    pl.kernel,
    out_type=jax.ShapeDtypeStruct((NC, NP, 16), f32),
    mesh=_mesh,
    scratch_types=[
        pltpu.VMEM((B,), i32),
        pltpu.VMEM((B, 16), f32),
        pltpu.VMEM((ZB, 16), f32),
        pltpu.VMEM_SHARED((NP, 16), f32),
    ],
    compiler_params=_sc_params,
)
def _sc_deg(dst_hbm, out_hbm, didx, ones, zbuf, acc):
    cid = lax.axis_index("c")
    sid = lax.axis_index("s")
    wid = cid * NS + sid

    one = jnp.full((16,), 1.0, f32)

    @pl.loop(0, B)
    def _(r):
        ones[r] = one

    _zero_acc(zbuf, acc, sid, 16)
    plsc.subcore_barrier()

    @pl.loop(0, NB)
    def _(b):
        pltpu.sync_copy(dst_hbm.at[pl.ds(wid * E_W + b * B, B)], didx)
        pltpu.sync_copy(ones, acc.at[didx], add=True)

    plsc.subcore_barrier()
    _copy_out(acc, out_hbm, cid, sid)


def _make_sc_scatter(f):
    @functools.partial(
        pl.kernel,
        out_type=jax.ShapeDtypeStruct((NC, NP, f), f32),
        mesh=_mesh,
        scratch_types=[
            pltpu.VMEM((B,), i32),
            pltpu.VMEM((B,), i32),
            pltpu.VMEM((B, f), f32),
            pltpu.VMEM((ZB, f), f32),
            pltpu.VMEM_SHARED((NP, f), f32),
        ],
        compiler_params=_sc_params,
    )
    def _sc_scatter(g_hbm, src_hbm, dst_hbm, out_hbm, sidx, didx, rows, zbuf, acc):
        cid = lax.axis_index("c")
        sid = lax.axis_index("s")
        wid = cid * NS + sid

        _zero_acc(zbuf, acc, sid, f)
        plsc.subcore_barrier()

        @pl.loop(0, NB)
        def _(b):
            off = wid * E_W + b * B
            pltpu.sync_copy(src_hbm.at[pl.ds(off, B)], sidx)
            pltpu.sync_copy(dst_hbm.at[pl.ds(off, B)], didx)
            pltpu.sync_copy(g_hbm.at[sidx], rows)
            pltpu.sync_copy(rows, acc.at[didx], add=True)

        plsc.subcore_barrier()
        _copy_out(acc, out_hbm, cid, sid)

    return _sc_scatter


_sc_scatter_hid = _make_sc_scatter(HID_DIM)
_sc_scatter_emb = _make_sc_scatter(EMB_DIM)


def _mm1_body(x_ref, w_ref, o_ref):
    o_ref[...] = jnp.dot(x_ref[...], w_ref[...], preferred_element_type=f32)


_mm1 = pl.pallas_call(
    _mm1_body,
    out_shape=jax.ShapeDtypeStruct((N_NODES, HID_DIM), f32),
)


def _prep_body(degp_ref, xw_ref, g1_ref, dinv_ref):
    deg = jnp.sum(degp_ref[:, :N_NODES, :], axis=(0, 2)) + 1.0
    dinv = 1.0 / jnp.sqrt(deg)
    dinv_ref[...] = dinv[:, None]
    g1_ref[...] = dinv[:, None] * xw_ref[...]


_prep = pl.pallas_call(
    _prep_body,
    out_shape=(
        jax.ShapeDtypeStruct((N_NODES, HID_DIM), f32),
        jax.ShapeDtypeStruct((N_NODES, 1), f32),
    ),
)


def _mid_body(acc_ref, g1_ref, dinv_ref, b1_ref, w2_ref, g2_ref):
    dinv = dinv_ref[...]
    pre = dinv * (acc_ref[0, :N_NODES] + acc_ref[1, :N_NODES] + g1_ref[...]) + b1_ref[...]
    h1 = jnp.maximum(pre, 0.0)
    g2_ref[...] = dinv * jnp.dot(h1, w2_ref[...], preferred_element_type=f32)


_mid = pl.pallas_call(
    _mid_body,
    out_shape=jax.ShapeDtypeStruct((N_NODES, EMB_DIM), f32),
)


def _final_body(acc_ref, g2_ref, dinv_ref, b2_ref, o_ref):
    h2 = dinv_ref[...] * (acc_ref[0, :N_NODES] + acc_ref[1, :N_NODES] + g2_ref[...]) + b2_ref[...]
    nrm = jnp.sqrt(jnp.sum(h2 * h2, axis=1, keepdims=True))
    o_ref[...] = h2 / jnp.maximum(nrm, 1e-12)


_final = pl.pallas_call(
    _final_body,
    out_shape=jax.ShapeDtypeStruct((N_NODES, EMB_DIM), f32),
)


def kernel(x, edge_index, W1, b1, W2, b2):
    src = edge_index[0]
    dst = edge_index[1]
    xw1 = _mm1(x, W1)
    degp = _sc_deg(dst)
    g1, dinv = _prep(degp, xw1)
    acc1 = _sc_scatter_hid(g1, src, dst)
    g2 = _mid(acc1, g1, dinv, b1.reshape(1, HID_DIM), W2)
    acc2 = _sc_scatter_emb(g2, src, dst)
    return _final(acc2, g2, dinv, b2.reshape(1, EMB_DIM))
